# baseline (speedup 1.0000x reference)
import jax
import jax.numpy as jnp
from jax import lax
from jax.experimental import pallas as pl
from jax.experimental.pallas import tpu as pltpu
from jax.experimental.pallas import tpu_sc as plsc

_B = 1024
_L = 20
_N = 50
_H = 32
_NW = 32
_HR = _B * _L
_HPW = _HR // _NW
_CH = 128
_UPW = _B // _NW
_W = 128


def _sc_gather_body(hist_ref, uid_ref, iid_ref, utab_ref, itab_ref,
                    he_ref, ue_ref, ie_ref,
                    idx_v, rows_v, idx_s, rows_s):
    wid = lax.axis_index("s") * 2 + lax.axis_index("c")
    hb = wid * _HPW
    pltpu.sync_copy(hist_ref.at[pl.ds(hb, _HPW)], idx_v)
    for j in range(_HPW // _CH):
        pltpu.sync_copy(itab_ref.at[idx_v.at[pl.ds(j * _CH, _CH)]],
                        rows_v.at[pl.ds(j * _CH, _CH)])
    pltpu.sync_copy(rows_v, he_ref.at[pl.ds(hb, _HPW)])
    ub = wid * _UPW
    pltpu.sync_copy(uid_ref.at[pl.ds(ub, _UPW)], idx_s)
    pltpu.sync_copy(utab_ref.at[idx_s], rows_s)
    pltpu.sync_copy(rows_s, ue_ref.at[pl.ds(ub, _UPW)])
    pltpu.sync_copy(iid_ref.at[pl.ds(ub, _UPW)], idx_s)
    pltpu.sync_copy(itab_ref.at[idx_s], rows_s)
    pltpu.sync_copy(rows_s, ie_ref.at[pl.ds(ub, _UPW)])


def _sc_gather(hist, uids, iids, utab, itab):
    mesh = plsc.VectorSubcoreMesh(core_axis_name="c", subcore_axis_name="s")
    f = pl.kernel(
        _sc_gather_body,
        out_type=(
            jax.ShapeDtypeStruct((_HR, _W), jnp.float32),
            jax.ShapeDtypeStruct((_B, _W), jnp.float32),
            jax.ShapeDtypeStruct((_B, _W), jnp.float32),
        ),
        mesh=mesh,
        scratch_types=[
            pltpu.VMEM((_HPW,), jnp.int32),
            pltpu.VMEM((_HPW, _W), jnp.float32),
            pltpu.VMEM((_UPW,), jnp.int32),
            pltpu.VMEM((_UPW, _W), jnp.float32),
        ],
    )
    return f(hist, uids, iids, utab, itab)


def _tc_body(nb_ref, he_ref, ue_ref, ie_ref,
             wa_ref, wb_ref, ngb_ref, mt_ref, mb_ref,
             w1u_ref, w1a_ref, w1i_ref, b1_ref, w2_ref, b2_ref,
             ow_ref, ob_ref, out_ref, acc_ar, acc_he):
    i = pl.program_id(0)
    nb = nb_ref[0]
    he = he_ref[0]
    ha = jnp.sum(he * wa_ref[...], axis=0, keepdims=True)
    s = jnp.sum(nb * wb_ref[...], axis=1) + ha + ngb_ref[...]
    e = jnp.exp(jnp.maximum(s, 0.0))
    den = jnp.sum(e, axis=0, keepdims=True)
    unnorm = jnp.sum(e[:, None, :] * nb, axis=0)
    ar = unnorm * (1.0 / den)

    @pl.when(i == 0)
    def _():
        acc_ar[...] = ar
        acc_he[...] = he

    @pl.when(i > 0)
    def _():
        acc_ar[...] += ar
        acc_he[...] += he

    @pl.when(i == _L - 1)
    def _():
        arm = acc_ar[...] * (1.0 / _L)
        hem = acc_he[...] * (1.0 / _L)
        agg = hem + jnp.dot(mt_ref[...], arm,
                            preferred_element_type=jnp.float32) + mb_ref[...]
        h1 = jnp.maximum(
            jnp.dot(w1u_ref[...], ue_ref[...], preferred_element_type=jnp.float32)
            + jnp.dot(w1a_ref[...], agg, preferred_element_type=jnp.float32)
            + jnp.dot(w1i_ref[...], ie_ref[...], preferred_element_type=jnp.float32)
            + b1_ref[...], 0.0)
        h2 = jnp.maximum(
            jnp.dot(w2_ref[...], h1, preferred_element_type=jnp.float32)
            + b2_ref[...], 0.0)
        logit = jnp.sum(h2 * ow_ref[...], axis=0, keepdims=True) + ob_ref[...]
        out_ref[...] = 1.0 / (1.0 + jnp.exp(-logit))


def _tc_attention_mlp(nbt, het, uet, iet, wa, wb, ngb, mt, mb,
                      w1u, w1a, w1i, b1, w2, b2, ow, ob):
    def full(shp):
        return pl.BlockSpec(shp, lambda i: (0,) * len(shp))

    return pl.pallas_call(
        _tc_body,
        grid=(_L,),
        in_specs=[
            pl.BlockSpec((1, _N, _H, _B), lambda i: (i, 0, 0, 0)),
            pl.BlockSpec((1, _H, _B), lambda i: (i, 0, 0)),
            full((_H, _B)),
            full((_H, _B)),
            full((_H, 1)), full((_H, 1)), full((1, 1)),
            full((_H, _H)), full((_H, 1)),
            full((_H, _H)), full((_H, _H)), full((_H, _H)), full((_H, 1)),
            full((_H, _H)), full((_H, 1)),
            full((_H, 1)), full((1, 1)),
        ],
        out_specs=pl.BlockSpec((1, _B), lambda i: (0, 0)),
        out_shape=jax.ShapeDtypeStruct((1, _B), jnp.float32),
        scratch_shapes=[
            pltpu.VMEM((_H, _B), jnp.float32),
            pltpu.VMEM((_H, _B), jnp.float32),
        ],
    )(nbt, het, uet, iet, wa, wb, ngb, mt, mb, w1u, w1a, w1i, b1, w2, b2, ow, ob)


def kernel(user_ids, item_ids, history, neighbor_emb, user_table, item_table,
           fc1_W, fc1_b, fc2_W, fc2_b, out_W, out_b,
           rg_W, rg_b, ng_W, ng_b, mh_W, mh_b):
    _Q = _W // _H
    hist = history.reshape(-1).astype(jnp.int32)
    uids = user_ids.astype(jnp.int32)
    iids = item_ids.astype(jnp.int32)
    utab = user_table.reshape(-1, _W)
    itab = item_table.reshape(-1, _W)
    he_wide, ue_wide, ie_wide = _sc_gather(hist // _Q, uids // _Q, iids // _Q,
                                           utab, itab)
    he4 = he_wide.reshape(_B, _L, _Q, _H)
    he3 = jnp.take_along_axis(he4, (hist % _Q).reshape(_B, _L, 1, 1),
                              axis=2)[:, :, 0]
    ue = jnp.take_along_axis(ue_wide.reshape(_B, _Q, _H),
                             (uids % _Q).reshape(_B, 1, 1), axis=1)[:, 0]
    ie = jnp.take_along_axis(ie_wide.reshape(_B, _Q, _H),
                             (iids % _Q).reshape(_B, 1, 1), axis=1)[:, 0]
    nbt = neighbor_emb.transpose(1, 2, 3, 0)
    het = he3.transpose(1, 2, 0)
    uet = ue.T
    iet = ie.T

    wa = ng_W[:, :_H].T
    wb = ng_W[:, _H:].T
    ngb = ng_b.reshape(1, 1)
    mt = mh_W.reshape(_H, _H)
    mb = mh_b.reshape(_H, 1)
    w1u = fc1_W[:, :_H]
    w1a = fc1_W[:, _H:2 * _H]
    w1i = fc1_W[:, 2 * _H:]
    b1 = fc1_b.reshape(_H, 1)
    w2 = fc2_W
    b2 = fc2_b.reshape(_H, 1)
    ow = out_W.reshape(_H, 1)
    ob = out_b.reshape(1, 1)

    out2 = _tc_attention_mlp(nbt, het, uet, iet, wa, wb, ngb, mt, mb,
                             w1u, w1a, w1i, b1, w2, b2, ow, ob)
    return out2.reshape(_B)


# speedup vs baseline: 1.3073x; 1.3073x over previous
import jax
import jax.numpy as jnp
from jax import lax
from jax.experimental import pallas as pl
from jax.experimental.pallas import tpu as pltpu
from jax.experimental.pallas import tpu_sc as plsc

_B = 1024
_L = 20
_N = 50
_H = 32
_NW = 32
_HR = _B * _L
_HPW = _HR // _NW
_CH = 128
_UPW = _B // _NW
_W = 128


def _sc_gather_body(hist_ref, uid_ref, iid_ref, utab_ref, itab_ref,
                    he_ref, ue_ref, ie_ref,
                    idx_v, rows_v, idx_s, rows_s):
    wid = lax.axis_index("s") * 2 + lax.axis_index("c")
    hb = wid * _HPW
    pltpu.sync_copy(hist_ref.at[pl.ds(hb, _HPW)], idx_v)
    for j in range(_HPW // _CH):
        pltpu.sync_copy(itab_ref.at[idx_v.at[pl.ds(j * _CH, _CH)]],
                        rows_v.at[pl.ds(j * _CH, _CH)])
    pltpu.sync_copy(rows_v, he_ref.at[pl.ds(hb, _HPW)])
    ub = wid * _UPW
    pltpu.sync_copy(uid_ref.at[pl.ds(ub, _UPW)], idx_s)
    pltpu.sync_copy(utab_ref.at[idx_s], rows_s)
    pltpu.sync_copy(rows_s, ue_ref.at[pl.ds(ub, _UPW)])
    pltpu.sync_copy(iid_ref.at[pl.ds(ub, _UPW)], idx_s)
    pltpu.sync_copy(itab_ref.at[idx_s], rows_s)
    pltpu.sync_copy(rows_s, ie_ref.at[pl.ds(ub, _UPW)])


def _sc_gather(hist, uids, iids, utab, itab):
    mesh = plsc.VectorSubcoreMesh(core_axis_name="c", subcore_axis_name="s")
    f = pl.kernel(
        _sc_gather_body,
        out_type=(
            jax.ShapeDtypeStruct((_HR, _W), jnp.float32),
            jax.ShapeDtypeStruct((_B, _W), jnp.float32),
            jax.ShapeDtypeStruct((_B, _W), jnp.float32),
        ),
        mesh=mesh,
        scratch_types=[
            pltpu.VMEM((_HPW,), jnp.int32),
            pltpu.VMEM((_HPW, _W), jnp.float32),
            pltpu.VMEM((_UPW,), jnp.int32),
            pltpu.VMEM((_UPW, _W), jnp.float32),
        ],
    )
    return f(hist, uids, iids, utab, itab)


def _tc_body(nb_ref, he_ref, ue_ref, ie_ref,
             wa_ref, wsel_ref, ngb_ref, mt_ref, mb_ref,
             w1u_ref, w1a_ref, w1i_ref, b1_ref, w2_ref, b2_ref,
             ow_ref, ob_ref, out_ref, acc_ar, acc_he):
    i = pl.program_id(0)
    nb = nb_ref[0]
    he = he_ref[0]
    ha = jnp.sum(he * wa_ref[...], axis=0, keepdims=True)
    nb2 = nb.reshape(_N * _H, _B)
    s = jnp.dot(wsel_ref[...], nb2, preferred_element_type=jnp.float32) \
        + ha + ngb_ref[...]
    e = jnp.exp(jnp.maximum(s, 0.0))
    den = jnp.sum(e, axis=0, keepdims=True)
    unnorm = jnp.sum(e[:, None, :] * nb, axis=0)
    ar = unnorm * (1.0 / den)

    @pl.when(i == 0)
    def _():
        acc_ar[...] = ar
        acc_he[...] = he

    @pl.when(i > 0)
    def _():
        acc_ar[...] += ar
        acc_he[...] += he

    @pl.when(i == _L - 1)
    def _():
        arm = acc_ar[...] * (1.0 / _L)
        hem = acc_he[...] * (1.0 / _L)
        agg = hem + jnp.dot(mt_ref[...], arm,
                            preferred_element_type=jnp.float32) + mb_ref[...]
        h1 = jnp.maximum(
            jnp.dot(w1u_ref[...], ue_ref[...], preferred_element_type=jnp.float32)
            + jnp.dot(w1a_ref[...], agg, preferred_element_type=jnp.float32)
            + jnp.dot(w1i_ref[...], ie_ref[...], preferred_element_type=jnp.float32)
            + b1_ref[...], 0.0)
        h2 = jnp.maximum(
            jnp.dot(w2_ref[...], h1, preferred_element_type=jnp.float32)
            + b2_ref[...], 0.0)
        logit = jnp.sum(h2 * ow_ref[...], axis=0, keepdims=True) + ob_ref[...]
        out_ref[...] = 1.0 / (1.0 + jnp.exp(-logit))


def _tc_attention_mlp(nbt, het, uet, iet, wa, wsel, ngb, mt, mb,
                      w1u, w1a, w1i, b1, w2, b2, ow, ob):
    def full(shp):
        return pl.BlockSpec(shp, lambda i: (0,) * len(shp))

    return pl.pallas_call(
        _tc_body,
        grid=(_L,),
        in_specs=[
            pl.BlockSpec((1, _N, _H, _B), lambda i: (i, 0, 0, 0)),
            pl.BlockSpec((1, _H, _B), lambda i: (i, 0, 0)),
            full((_H, _B)),
            full((_H, _B)),
            full((_H, 1)), full((_N, _N * _H)), full((1, 1)),
            full((_H, _H)), full((_H, 1)),
            full((_H, _H)), full((_H, _H)), full((_H, _H)), full((_H, 1)),
            full((_H, _H)), full((_H, 1)),
            full((_H, 1)), full((1, 1)),
        ],
        out_specs=pl.BlockSpec((1, _B), lambda i: (0, 0)),
        out_shape=jax.ShapeDtypeStruct((1, _B), jnp.float32),
        scratch_shapes=[
            pltpu.VMEM((_H, _B), jnp.float32),
            pltpu.VMEM((_H, _B), jnp.float32),
        ],
    )(nbt, het, uet, iet, wa, wsel, ngb, mt, mb, w1u, w1a, w1i, b1, w2, b2, ow, ob)


def kernel(user_ids, item_ids, history, neighbor_emb, user_table, item_table,
           fc1_W, fc1_b, fc2_W, fc2_b, out_W, out_b,
           rg_W, rg_b, ng_W, ng_b, mh_W, mh_b):
    hist = history.reshape(-1).astype(jnp.int32)
    utab = jnp.pad(user_table, ((0, 0), (0, _W - _H)))
    itab = jnp.pad(item_table, ((0, 0), (0, _W - _H)))
    he_flat, ue, ie = _sc_gather(hist, user_ids.astype(jnp.int32),
                                 item_ids.astype(jnp.int32), utab, itab)
    nbt = neighbor_emb.transpose(1, 2, 3, 0)
    het = he_flat.reshape(_B, _L, _W)[:, :, :_H].transpose(1, 2, 0)
    uet = ue[:, :_H].T
    iet = ie[:, :_H].T

    wa = ng_W[:, :_H].T
    wsel = jnp.kron(jnp.eye(_N, dtype=jnp.float32), ng_W[:, _H:])
    ngb = ng_b.reshape(1, 1)
    mt = mh_W.reshape(_H, _H)
    mb = mh_b.reshape(_H, 1)
    w1u = fc1_W[:, :_H]
    w1a = fc1_W[:, _H:2 * _H]
    w1i = fc1_W[:, 2 * _H:]
    b1 = fc1_b.reshape(_H, 1)
    w2 = fc2_W
    b2 = fc2_b.reshape(_H, 1)
    ow = out_W.reshape(_H, 1)
    ob = out_b.reshape(1, 1)

    out2 = _tc_attention_mlp(nbt, het, uet, iet, wa, wsel, ngb, mt, mb,
                             w1u, w1a, w1i, b1, w2, b2, ow, ob)
    return out2.reshape(_B)


# speedup vs baseline: 1.3381x; 1.0236x over previous
import jax
import jax.numpy as jnp
from jax import lax
from jax.experimental import pallas as pl
from jax.experimental.pallas import tpu as pltpu
from jax.experimental.pallas import tpu_sc as plsc

_B = 1024
_L = 20
_N = 50
_H = 32
_NW = 32
_HR = _B * _L
_HPW = _HR // _NW
_CH = 128
_UPW = _B // _NW
_W = 128


def _sc_gather_body(hist_ref, uid_ref, iid_ref, utab_ref, itab_ref,
                    he_ref, ue_ref, ie_ref,
                    idx_v, rows_v, idx_s, rows_s, idx_i, rows_i, sem):
    wid = lax.axis_index("s") * 2 + lax.axis_index("c")
    hb = wid * _HPW
    ub = wid * _UPW
    pltpu.sync_copy(hist_ref.at[pl.ds(hb, _HPW)], idx_v)
    pltpu.sync_copy(uid_ref.at[pl.ds(ub, _UPW)], idx_s)
    pltpu.sync_copy(iid_ref.at[pl.ds(ub, _UPW)], idx_i)
    cps = [pltpu.async_copy(itab_ref.at[idx_v.at[pl.ds(j * _CH, _CH)]],
                            rows_v.at[pl.ds(j * _CH, _CH)], sem)
           for j in range(_HPW // _CH)]
    cps.append(pltpu.async_copy(utab_ref.at[idx_s], rows_s, sem))
    cps.append(pltpu.async_copy(itab_ref.at[idx_i], rows_i, sem))
    for c in cps:
        c.wait()
    pltpu.sync_copy(rows_v, he_ref.at[pl.ds(hb, _HPW)])
    pltpu.sync_copy(rows_s, ue_ref.at[pl.ds(ub, _UPW)])
    pltpu.sync_copy(rows_i, ie_ref.at[pl.ds(ub, _UPW)])


def _sc_gather(hist, uids, iids, utab, itab):
    mesh = plsc.VectorSubcoreMesh(core_axis_name="c", subcore_axis_name="s")
    f = pl.kernel(
        _sc_gather_body,
        out_type=(
            jax.ShapeDtypeStruct((_HR, _W), jnp.float32),
            jax.ShapeDtypeStruct((_B, _W), jnp.float32),
            jax.ShapeDtypeStruct((_B, _W), jnp.float32),
        ),
        mesh=mesh,
        scratch_types=[
            pltpu.VMEM((_HPW,), jnp.int32),
            pltpu.VMEM((_HPW, _W), jnp.float32),
            pltpu.VMEM((_UPW,), jnp.int32),
            pltpu.VMEM((_UPW, _W), jnp.float32),
            pltpu.VMEM((_UPW,), jnp.int32),
            pltpu.VMEM((_UPW, _W), jnp.float32),
            pltpu.SemaphoreType.DMA,
        ],
    )
    return f(hist, uids, iids, utab, itab)


def _tc_body(nb_ref, he_ref, ue_ref, ie_ref,
             wa_ref, wsel_ref, ngb_ref, mt_ref, mb_ref,
             w1u_ref, w1a_ref, w1i_ref, b1_ref, w2_ref, b2_ref,
             ow_ref, ob_ref, out_ref, acc_ar, acc_he):
    i = pl.program_id(0)
    nb = nb_ref[0]
    he = he_ref[0]
    ha = jnp.sum(he * wa_ref[...], axis=0, keepdims=True)
    nb2 = nb.reshape(_N * _H, _B)
    s = jnp.dot(wsel_ref[...], nb2, preferred_element_type=jnp.float32) \
        + ha + ngb_ref[...]
    e = jnp.exp(jnp.maximum(s, 0.0))
    den = jnp.sum(e, axis=0, keepdims=True)
    unnorm = jnp.sum(e[:, None, :] * nb, axis=0)
    ar = unnorm * (1.0 / den)

    @pl.when(i == 0)
    def _():
        acc_ar[...] = ar
        acc_he[...] = he

    @pl.when(i > 0)
    def _():
        acc_ar[...] += ar
        acc_he[...] += he

    @pl.when(i == _L - 1)
    def _():
        arm = acc_ar[...] * (1.0 / _L)
        hem = acc_he[...] * (1.0 / _L)
        agg = hem + jnp.dot(mt_ref[...], arm,
                            preferred_element_type=jnp.float32) + mb_ref[...]
        h1 = jnp.maximum(
            jnp.dot(w1u_ref[...], ue_ref[...], preferred_element_type=jnp.float32)
            + jnp.dot(w1a_ref[...], agg, preferred_element_type=jnp.float32)
            + jnp.dot(w1i_ref[...], ie_ref[...], preferred_element_type=jnp.float32)
            + b1_ref[...], 0.0)
        h2 = jnp.maximum(
            jnp.dot(w2_ref[...], h1, preferred_element_type=jnp.float32)
            + b2_ref[...], 0.0)
        logit = jnp.sum(h2 * ow_ref[...], axis=0, keepdims=True) + ob_ref[...]
        out_ref[...] = 1.0 / (1.0 + jnp.exp(-logit))


def _tc_attention_mlp(nbt, het, uet, iet, wa, wsel, ngb, mt, mb,
                      w1u, w1a, w1i, b1, w2, b2, ow, ob):
    def full(shp):
        return pl.BlockSpec(shp, lambda i: (0,) * len(shp))

    return pl.pallas_call(
        _tc_body,
        grid=(_L,),
        in_specs=[
            pl.BlockSpec((1, _N, _H, _B), lambda i: (i, 0, 0, 0)),
            pl.BlockSpec((1, _H, _B), lambda i: (i, 0, 0)),
            full((_H, _B)),
            full((_H, _B)),
            full((_H, 1)), full((_N, _N * _H)), full((1, 1)),
            full((_H, _H)), full((_H, 1)),
            full((_H, _H)), full((_H, _H)), full((_H, _H)), full((_H, 1)),
            full((_H, _H)), full((_H, 1)),
            full((_H, 1)), full((1, 1)),
        ],
        out_specs=pl.BlockSpec((1, _B), lambda i: (0, 0)),
        out_shape=jax.ShapeDtypeStruct((1, _B), jnp.float32),
        scratch_shapes=[
            pltpu.VMEM((_H, _B), jnp.float32),
            pltpu.VMEM((_H, _B), jnp.float32),
        ],
    )(nbt, het, uet, iet, wa, wsel, ngb, mt, mb, w1u, w1a, w1i, b1, w2, b2, ow, ob)


def kernel(user_ids, item_ids, history, neighbor_emb, user_table, item_table,
           fc1_W, fc1_b, fc2_W, fc2_b, out_W, out_b,
           rg_W, rg_b, ng_W, ng_b, mh_W, mh_b):
    hist = history.reshape(-1).astype(jnp.int32)
    utab = jnp.pad(user_table, ((0, 0), (0, _W - _H)))
    itab = jnp.pad(item_table, ((0, 0), (0, _W - _H)))
    he_flat, ue, ie = _sc_gather(hist, user_ids.astype(jnp.int32),
                                 item_ids.astype(jnp.int32), utab, itab)
    nbt = neighbor_emb.transpose(1, 2, 3, 0)
    het = he_flat.reshape(_B, _L, _W)[:, :, :_H].transpose(1, 2, 0)
    uet = ue[:, :_H].T
    iet = ie[:, :_H].T

    wa = ng_W[:, :_H].T
    wsel = jnp.kron(jnp.eye(_N, dtype=jnp.float32), ng_W[:, _H:])
    ngb = ng_b.reshape(1, 1)
    mt = mh_W.reshape(_H, _H)
    mb = mh_b.reshape(_H, 1)
    w1u = fc1_W[:, :_H]
    w1a = fc1_W[:, _H:2 * _H]
    w1i = fc1_W[:, 2 * _H:]
    b1 = fc1_b.reshape(_H, 1)
    w2 = fc2_W
    b2 = fc2_b.reshape(_H, 1)
    ow = out_W.reshape(_H, 1)
    ob = out_b.reshape(1, 1)

    out2 = _tc_attention_mlp(nbt, het, uet, iet, wa, wsel, ngb, mt, mb,
                             w1u, w1a, w1i, b1, w2, b2, ow, ob)
    return out2.reshape(_B)
